# Initial kernel scaffold; baseline (speedup 1.0000x reference)
import jax
import jax.numpy as jnp
from jax.experimental import pallas as pl


def kernel(x, edge_index, eq_labels, batch, eq_table, eps0, W0a, b0a, W0b, b0b, g0, beta0, eps1, W1a, b1a, W1b, b1b, g1, beta1, eps2, W2a, b2a, W2b, b2b, g2, beta2, Wf, bf):
    raise NotImplementedError("write your pallas kernel here")


# speedup vs baseline: 5.1286x; 5.1286x over previous
import functools

import jax
import jax.numpy as jnp
from jax import lax
from jax.experimental import pallas as pl
from jax.experimental.pallas import tpu as pltpu
from jax.experimental.pallas import tpu_sc as plsc

N = 10000
E = 320000
D = 128
H = 64
NEQ = 16
EMB = 8
DIN = D + EMB

NC = 2
NS = 16
TILES = NC * NS
T = E // TILES
C = 80
K = T // C
N_PAD = 10240
RPS = N_PAD // NS
ZR = 128

BN = 1000
NB = N // BN


def _sc_agg_body(W, h_hbm, src_hbm, dst_hbm, out_hbm,
                 srcv, dstv, rows, zv, acc, sem):
    c = lax.axis_index("c")
    s = lax.axis_index("s")

    zero16 = jnp.zeros((16,), jnp.float32)
    offs = list(range(0, W - 15, 16))
    if offs[-1] + 16 < W:
        offs.append(W - 16)

    def zrow(i, carry):
        for o in offs:
            zv[i, pl.ds(o, 16)] = zero16
        return carry

    lax.fori_loop(0, ZR, zrow, None)
    for r in range(RPS // ZR):
        pltpu.sync_copy(zv, acc.at[pl.ds(s * RPS + r * ZR, ZR)])
    plsc.subcore_barrier()

    base = (c * NS + s) * T

    def chunk(j, carry):
        off = base + j * C
        pltpu.sync_copy(src_hbm.at[pl.ds(off, C)], srcv)
        pltpu.sync_copy(dst_hbm.at[pl.ds(off, C)], dstv)
        pltpu.async_copy(h_hbm.at[srcv], rows, sem).wait()
        pltpu.sync_copy(rows, acc.at[dstv], add=True)
        return carry

    lax.fori_loop(0, K, chunk, None)
    plsc.subcore_barrier()

    pltpu.sync_copy(acc.at[pl.ds(s * RPS, RPS)],
                    out_hbm.at[pl.ds(c * N_PAD + s * RPS, RPS)])


@functools.cache
def _make_sc_agg(W):
    return pl.kernel(
        functools.partial(_sc_agg_body, W),
        out_type=jax.ShapeDtypeStruct((2 * N_PAD, W), jnp.float32),
        mesh=plsc.VectorSubcoreMesh(core_axis_name="c", subcore_axis_name="s",
                                    num_cores=NC, num_subcores=NS),
        scratch_types=[
            pltpu.VMEM((C,), jnp.int32),
            pltpu.VMEM((C,), jnp.int32),
            pltpu.VMEM((C, W), jnp.float32),
            pltpu.VMEM((ZR, W), jnp.float32),
            pltpu.VMEM_SHARED((N_PAD, W), jnp.float32),
            pltpu.SemaphoreType.DMA,
        ],
        compiler_params=pltpu.CompilerParams(use_tc_tiling_on_sc=False),
    )


def _sc_agg(h, src, dst):
    return _make_sc_agg(h.shape[1])(h, src, dst)


def _h0_body(x_ref, eq_ref, tab_ref, o_ref):
    labels = eq_ref[0, 0, :]
    oh = (labels[:, None] ==
          lax.broadcasted_iota(jnp.int32, (1, NEQ), 1)).astype(jnp.float32)
    emb = jnp.dot(oh, tab_ref[...], preferred_element_type=jnp.float32,
                  precision=lax.Precision.HIGHEST)
    o_ref[...] = jnp.concatenate([x_ref[...], emb], axis=1)


def _h0(x, eq_labels, eq_table):
    eq3 = eq_labels.reshape(NB, 1, BN)
    return pl.pallas_call(
        _h0_body,
        grid=(NB,),
        in_specs=[
            pl.BlockSpec((BN, D), lambda b: (b, 0)),
            pl.BlockSpec((1, 1, BN), lambda b: (b, 0, 0)),
            pl.BlockSpec((NEQ, EMB), lambda b: (0, 0)),
        ],
        out_specs=pl.BlockSpec((BN, DIN), lambda b: (b, 0)),
        out_shape=jax.ShapeDtypeStruct((N, DIN), jnp.float32),
    )(x, eq3, eq_table)


def _mlp_body(eps_ref, h_ref, parts_ref, wa_ref, ba_ref, wb_ref, bb_ref,
              z_ref, s_ref):
    b = pl.program_id(0)
    agg = parts_ref[0] + parts_ref[1]
    z = (1.0 + eps_ref[0]) * h_ref[...] + agg
    a = jnp.maximum(
        jnp.dot(z, wa_ref[...], preferred_element_type=jnp.float32)
        + ba_ref[...], 0.0)
    z2 = jnp.maximum(
        jnp.dot(a, wb_ref[...], preferred_element_type=jnp.float32)
        + bb_ref[...], 0.0)
    z_ref[...] = z2

    @pl.when(b == 0)
    def _init():
        s_ref[...] = jnp.zeros_like(s_ref)

    s_ref[...] += jnp.sum(z2, axis=0, keepdims=True)


def _mlp(eps, h, parts, Wa, ba, Wb, bb):
    Wd = h.shape[1]
    parts3 = parts.reshape(2, N_PAD, Wd)
    return pl.pallas_call(
        _mlp_body,
        grid=(NB,),
        in_specs=[
            pl.BlockSpec(memory_space=pltpu.SMEM),
            pl.BlockSpec((BN, Wd), lambda b: (b, 0)),
            pl.BlockSpec((2, BN, Wd), lambda b: (0, b, 0)),
            pl.BlockSpec((Wd, H), lambda b: (0, 0)),
            pl.BlockSpec((1, H), lambda b: (0, 0)),
            pl.BlockSpec((H, H), lambda b: (0, 0)),
            pl.BlockSpec((1, H), lambda b: (0, 0)),
        ],
        out_specs=[
            pl.BlockSpec((BN, H), lambda b: (b, 0)),
            pl.BlockSpec((1, H), lambda b: (0, 0)),
        ],
        out_shape=[
            jax.ShapeDtypeStruct((N, H), jnp.float32),
            jax.ShapeDtypeStruct((1, H), jnp.float32),
        ],
    )(jnp.reshape(eps, (1,)), h, parts3, Wa, ba.reshape(1, H), Wb,
      bb.reshape(1, H))


def _bn_body(z_ref, s_ref, g_ref, be_ref, o_ref, ssq_ref):
    ph = pl.program_id(0)
    b = pl.program_id(1)
    mu = s_ref[...] / N

    @pl.when(ph == 0)
    def _acc():
        @pl.when(b == 0)
        def _init():
            ssq_ref[...] = jnp.zeros_like(ssq_ref)
        d = z_ref[...] - mu
        ssq_ref[...] += jnp.sum(d * d, axis=0, keepdims=True)

    @pl.when(ph == 1)
    def _norm():
        var = ssq_ref[...] / N
        zn = ((z_ref[...] - mu) / jnp.sqrt(var + 1e-5) * g_ref[...]
              + be_ref[...])
        o_ref[...] = jnp.maximum(zn, 0.0)


def _bn(z, s, g, be):
    return pl.pallas_call(
        _bn_body,
        grid=(2, NB),
        in_specs=[
            pl.BlockSpec((BN, H), lambda ph, b: (b, 0)),
            pl.BlockSpec((1, H), lambda ph, b: (0, 0)),
            pl.BlockSpec((1, H), lambda ph, b: (0, 0)),
            pl.BlockSpec((1, H), lambda ph, b: (0, 0)),
        ],
        out_specs=pl.BlockSpec((BN, H), lambda ph, b: (b * ph, 0)),
        out_shape=jax.ShapeDtypeStruct((N, H), jnp.float32),
        scratch_shapes=[pltpu.VMEM((1, H), jnp.float32)],
    )(z, s, g.reshape(1, H), be.reshape(1, H))


def _bn_readout_body(z_ref, s_ref, g_ref, be_ref, wf_ref, bf_ref,
                     o_ref, ssq_ref):
    ph = pl.program_id(0)
    b = pl.program_id(1)
    mu = s_ref[...] / N

    @pl.when(ph == 0)
    def _acc():
        @pl.when(b == 0)
        def _init():
            ssq_ref[...] = jnp.zeros_like(ssq_ref)
        d = z_ref[...] - mu
        ssq_ref[...] += jnp.sum(d * d, axis=0, keepdims=True)

    @pl.when(ph == 1)
    def _norm():
        var = ssq_ref[...] / N
        zn = ((z_ref[...] - mu) / jnp.sqrt(var + 1e-5) * g_ref[...]
              + be_ref[...])
        hh = jnp.maximum(zn, 0.0)
        o_ref[...] = (jnp.dot(hh, wf_ref[...],
                              preferred_element_type=jnp.float32)
                      + bf_ref[...])


def _bn_readout(z, s, g, be, Wf, bf):
    return pl.pallas_call(
        _bn_readout_body,
        grid=(2, NB),
        in_specs=[
            pl.BlockSpec((BN, H), lambda ph, b: (b, 0)),
            pl.BlockSpec((1, H), lambda ph, b: (0, 0)),
            pl.BlockSpec((1, H), lambda ph, b: (0, 0)),
            pl.BlockSpec((1, H), lambda ph, b: (0, 0)),
            pl.BlockSpec((H, 1), lambda ph, b: (0, 0)),
            pl.BlockSpec((1, 1), lambda ph, b: (0, 0)),
        ],
        out_specs=pl.BlockSpec((BN, 1), lambda ph, b: (b * ph, 0)),
        out_shape=jax.ShapeDtypeStruct((N, 1), jnp.float32),
        scratch_shapes=[pltpu.VMEM((1, H), jnp.float32)],
    )(z, s, g.reshape(1, H), be.reshape(1, H), Wf, bf.reshape(1, 1))


def kernel(x, edge_index, eq_labels, batch, eq_table, eps0, W0a, b0a, W0b,
           b0b, g0, beta0, eps1, W1a, b1a, W1b, b1b, g1, beta1, eps2, W2a,
           b2a, W2b, b2b, g2, beta2, Wf, bf):
    src, dst = edge_index[0], edge_index[1]

    h = _h0(x, eq_labels, eq_table)

    layers = [
        (eps0, W0a, b0a, W0b, b0b, g0, beta0),
        (eps1, W1a, b1a, W1b, b1b, g1, beta1),
        (eps2, W2a, b2a, W2b, b2b, g2, beta2),
    ]

    for i, (eps, Wa, ba, Wb, bb, g, be) in enumerate(layers):
        parts = _sc_agg(h, src, dst)
        z, s = _mlp(eps, h, parts, Wa, ba, Wb, bb)
        if i < 2:
            h = _bn(z, s, g, be)
        else:
            out = _bn_readout(z, s, g, be, Wf, bf)

    return out.reshape(N)


# speedup vs baseline: 10.6626x; 2.0791x over previous
import functools

import jax
import jax.numpy as jnp
from jax import lax
from jax.experimental import pallas as pl
from jax.experimental.pallas import tpu as pltpu
from jax.experimental.pallas import tpu_sc as plsc

N = 10000
E = 320000
D = 128
H = 64
NEQ = 16
EMB = 8
DIN = D + EMB

NC = 2
NS = 16
TILES = NC * NS
T = E // TILES
C = 80
K = T // C
N_PAD = 10240
RPS = N_PAD // NS
ZR = 128

BN = 1000
NB = N // BN


def _sc_agg_body(W, h_hbm, src_hbm, dst_hbm, out_hbm,
                 srcall, dstall, rows0, rows1, acc, gs0, gs1):
    c = lax.axis_index("c")
    s = lax.axis_index("s")

    tile = c * NS + s
    pltpu.sync_copy(src_hbm.at[pl.ds(tile * K, K)], srcall)
    pltpu.sync_copy(dst_hbm.at[pl.ds(tile * K, K)], dstall)

    zero16 = jnp.zeros((16,), jnp.float32)
    offs = list(range(0, W - 15, 16))
    if offs[-1] + 16 < W:
        offs.append(W - 16)

    def zrow(i, carry):
        for o in offs:
            rows0[i, pl.ds(o, 16)] = zero16
        return carry

    lax.fori_loop(0, C, zrow, None)
    for r in range(RPS // C):
        pltpu.sync_copy(rows0, acc.at[pl.ds(s * RPS + r * C, C)])
    plsc.subcore_barrier()

    pltpu.async_copy(h_hbm.at[srcall.at[0]], rows0, gs0)

    def group(g, carry):
        t0 = 2 * g
        pltpu.async_copy(h_hbm.at[srcall.at[t0 + 1]], rows1, gs1)
        pltpu.make_async_copy(h_hbm.at[srcall.at[t0]], rows0, gs0).wait()
        pltpu.sync_copy(rows0, acc.at[dstall.at[t0]], add=True)
        pltpu.async_copy(h_hbm.at[srcall.at[t0 + 2]], rows0, gs0)
        pltpu.make_async_copy(h_hbm.at[srcall.at[t0 + 1]], rows1, gs1).wait()
        pltpu.sync_copy(rows1, acc.at[dstall.at[t0 + 1]], add=True)
        return carry

    lax.fori_loop(0, (K - 1) // 2, group, None)
    pltpu.make_async_copy(h_hbm.at[srcall.at[K - 1]], rows0, gs0).wait()
    pltpu.sync_copy(rows0, acc.at[dstall.at[K - 1]], add=True)
    plsc.subcore_barrier()

    pltpu.sync_copy(acc.at[pl.ds(s * RPS, RPS)],
                    out_hbm.at[pl.ds(c * N_PAD + s * RPS, RPS)])


@functools.cache
def _make_sc_agg(W):
    return pl.kernel(
        functools.partial(_sc_agg_body, W),
        out_type=jax.ShapeDtypeStruct((2 * N_PAD, W), jnp.float32),
        mesh=plsc.VectorSubcoreMesh(core_axis_name="c", subcore_axis_name="s",
                                    num_cores=NC, num_subcores=NS),
        scratch_types=[
            pltpu.VMEM((K, C), jnp.int32),
            pltpu.VMEM((K, C), jnp.int32),
            pltpu.VMEM((C, W), jnp.float32),
            pltpu.VMEM((C, W), jnp.float32),
            pltpu.VMEM_SHARED((N_PAD, W), jnp.float32),
            pltpu.SemaphoreType.DMA,
            pltpu.SemaphoreType.DMA,
        ],
        compiler_params=pltpu.CompilerParams(use_tc_tiling_on_sc=False),
    )


def _sc_agg(h, src, dst):
    return _make_sc_agg(h.shape[1])(h, src.reshape(TILES * K, C),
                                    dst.reshape(TILES * K, C))


def _h0_body(x_ref, eq_ref, tab_ref, o_ref):
    labels = eq_ref[0, 0, :]
    oh = (labels[:, None] ==
          lax.broadcasted_iota(jnp.int32, (1, NEQ), 1)).astype(jnp.float32)
    emb = jnp.dot(oh, tab_ref[...], preferred_element_type=jnp.float32,
                  precision=lax.Precision.HIGHEST)
    o_ref[...] = jnp.concatenate([x_ref[...], emb], axis=1)


def _h0(x, eq_labels, eq_table):
    eq3 = eq_labels.reshape(NB, 1, BN)
    return pl.pallas_call(
        _h0_body,
        grid=(NB,),
        in_specs=[
            pl.BlockSpec((BN, D), lambda b: (b, 0)),
            pl.BlockSpec((1, 1, BN), lambda b: (b, 0, 0)),
            pl.BlockSpec((NEQ, EMB), lambda b: (0, 0)),
        ],
        out_specs=pl.BlockSpec((BN, DIN), lambda b: (b, 0)),
        out_shape=jax.ShapeDtypeStruct((N, DIN), jnp.float32),
    )(x, eq3, eq_table)


def _mlp_body(eps_ref, h_ref, parts_ref, wa_ref, ba_ref, wb_ref, bb_ref,
              z_ref, s_ref):
    b = pl.program_id(0)
    agg = parts_ref[0] + parts_ref[1]
    z = (1.0 + eps_ref[0]) * h_ref[...] + agg
    a = jnp.maximum(
        jnp.dot(z, wa_ref[...], preferred_element_type=jnp.float32)
        + ba_ref[...], 0.0)
    z2 = jnp.maximum(
        jnp.dot(a, wb_ref[...], preferred_element_type=jnp.float32)
        + bb_ref[...], 0.0)
    z_ref[...] = z2

    @pl.when(b == 0)
    def _init():
        s_ref[...] = jnp.zeros_like(s_ref)

    s_ref[...] += jnp.sum(z2, axis=0, keepdims=True)


def _mlp(eps, h, parts, Wa, ba, Wb, bb):
    Wd = h.shape[1]
    parts3 = parts.reshape(2, N_PAD, Wd)
    return pl.pallas_call(
        _mlp_body,
        grid=(NB,),
        in_specs=[
            pl.BlockSpec(memory_space=pltpu.SMEM),
            pl.BlockSpec((BN, Wd), lambda b: (b, 0)),
            pl.BlockSpec((2, BN, Wd), lambda b: (0, b, 0)),
            pl.BlockSpec((Wd, H), lambda b: (0, 0)),
            pl.BlockSpec((1, H), lambda b: (0, 0)),
            pl.BlockSpec((H, H), lambda b: (0, 0)),
            pl.BlockSpec((1, H), lambda b: (0, 0)),
        ],
        out_specs=[
            pl.BlockSpec((BN, H), lambda b: (b, 0)),
            pl.BlockSpec((1, H), lambda b: (0, 0)),
        ],
        out_shape=[
            jax.ShapeDtypeStruct((N, H), jnp.float32),
            jax.ShapeDtypeStruct((1, H), jnp.float32),
        ],
    )(jnp.reshape(eps, (1,)), h, parts3, Wa, ba.reshape(1, H), Wb,
      bb.reshape(1, H))


def _bn_body(z_ref, s_ref, g_ref, be_ref, o_ref, ssq_ref):
    ph = pl.program_id(0)
    b = pl.program_id(1)
    mu = s_ref[...] / N

    @pl.when(ph == 0)
    def _acc():
        @pl.when(b == 0)
        def _init():
            ssq_ref[...] = jnp.zeros_like(ssq_ref)
        d = z_ref[...] - mu
        ssq_ref[...] += jnp.sum(d * d, axis=0, keepdims=True)

    @pl.when(ph == 1)
    def _norm():
        var = ssq_ref[...] / N
        zn = ((z_ref[...] - mu) / jnp.sqrt(var + 1e-5) * g_ref[...]
              + be_ref[...])
        o_ref[...] = jnp.maximum(zn, 0.0)


def _bn(z, s, g, be):
    return pl.pallas_call(
        _bn_body,
        grid=(2, NB),
        in_specs=[
            pl.BlockSpec((BN, H), lambda ph, b: (b, 0)),
            pl.BlockSpec((1, H), lambda ph, b: (0, 0)),
            pl.BlockSpec((1, H), lambda ph, b: (0, 0)),
            pl.BlockSpec((1, H), lambda ph, b: (0, 0)),
        ],
        out_specs=pl.BlockSpec((BN, H), lambda ph, b: (b * ph, 0)),
        out_shape=jax.ShapeDtypeStruct((N, H), jnp.float32),
        scratch_shapes=[pltpu.VMEM((1, H), jnp.float32)],
    )(z, s, g.reshape(1, H), be.reshape(1, H))


def _bn_readout_body(z_ref, s_ref, g_ref, be_ref, wf_ref, bf_ref,
                     o_ref, ssq_ref):
    ph = pl.program_id(0)
    b = pl.program_id(1)
    mu = s_ref[...] / N

    @pl.when(ph == 0)
    def _acc():
        @pl.when(b == 0)
        def _init():
            ssq_ref[...] = jnp.zeros_like(ssq_ref)
        d = z_ref[...] - mu
        ssq_ref[...] += jnp.sum(d * d, axis=0, keepdims=True)

    @pl.when(ph == 1)
    def _norm():
        var = ssq_ref[...] / N
        zn = ((z_ref[...] - mu) / jnp.sqrt(var + 1e-5) * g_ref[...]
              + be_ref[...])
        hh = jnp.maximum(zn, 0.0)
        o_ref[...] = (jnp.dot(hh, wf_ref[...],
                              preferred_element_type=jnp.float32)
                      + bf_ref[...])


def _bn_readout(z, s, g, be, Wf, bf):
    return pl.pallas_call(
        _bn_readout_body,
        grid=(2, NB),
        in_specs=[
            pl.BlockSpec((BN, H), lambda ph, b: (b, 0)),
            pl.BlockSpec((1, H), lambda ph, b: (0, 0)),
            pl.BlockSpec((1, H), lambda ph, b: (0, 0)),
            pl.BlockSpec((1, H), lambda ph, b: (0, 0)),
            pl.BlockSpec((H, 1), lambda ph, b: (0, 0)),
            pl.BlockSpec((1, 1), lambda ph, b: (0, 0)),
        ],
        out_specs=pl.BlockSpec((BN, 1), lambda ph, b: (b * ph, 0)),
        out_shape=jax.ShapeDtypeStruct((N, 1), jnp.float32),
        scratch_shapes=[pltpu.VMEM((1, H), jnp.float32)],
    )(z, s, g.reshape(1, H), be.reshape(1, H), Wf, bf.reshape(1, 1))


def kernel(x, edge_index, eq_labels, batch, eq_table, eps0, W0a, b0a, W0b,
           b0b, g0, beta0, eps1, W1a, b1a, W1b, b1b, g1, beta1, eps2, W2a,
           b2a, W2b, b2b, g2, beta2, Wf, bf):
    src, dst = edge_index[0], edge_index[1]

    h = _h0(x, eq_labels, eq_table)

    layers = [
        (eps0, W0a, b0a, W0b, b0b, g0, beta0),
        (eps1, W1a, b1a, W1b, b1b, g1, beta1),
        (eps2, W2a, b2a, W2b, b2b, g2, beta2),
    ]

    for i, (eps, Wa, ba, Wb, bb, g, be) in enumerate(layers):
        parts = _sc_agg(h, src, dst)
        z, s = _mlp(eps, h, parts, Wa, ba, Wb, bb)
        if i < 2:
            h = _bn(z, s, g, be)
        else:
            out = _bn_readout(z, s, g, be, Wf, bf)

    return out.reshape(N)


# speedup vs baseline: 11.2521x; 1.0553x over previous
import functools

import jax
import jax.numpy as jnp
from jax import lax
from jax.experimental import pallas as pl
from jax.experimental.pallas import tpu as pltpu
from jax.experimental.pallas import tpu_sc as plsc

N = 10000
E = 320000
D = 128
H = 64
NEQ = 16
EMB = 8
DIN = D + EMB

NC = 2
NS = 16
TILES = NC * NS
T = E // TILES
C = 80
K = T // C
N_PAD = 10240
RPS = N_PAD // NS
ZR = 128

BN = 1000
NB = N // BN


def _sc_agg_body(W, h_hbm, src_hbm, dst_hbm, out_hbm,
                 srcall, dstall, rows0, rows1, acc, gs0, gs1):
    c = lax.axis_index("c")
    s = lax.axis_index("s")

    tile = c * NS + s
    pltpu.sync_copy(src_hbm.at[pl.ds(tile * K, K)], srcall)
    pltpu.sync_copy(dst_hbm.at[pl.ds(tile * K, K)], dstall)

    zero16 = jnp.zeros((16,), jnp.float32)
    offs = list(range(0, W - 15, 16))
    if offs[-1] + 16 < W:
        offs.append(W - 16)

    def zrow(i, carry):
        for o in offs:
            rows0[i, pl.ds(o, 16)] = zero16
        return carry

    lax.fori_loop(0, C, zrow, None)
    for r in range(RPS // C):
        pltpu.sync_copy(rows0, acc.at[pl.ds(s * RPS + r * C, C)])
    plsc.subcore_barrier()

    pltpu.async_copy(h_hbm.at[srcall.at[0]], rows0, gs0)

    def group(g, carry):
        t0 = 2 * g
        pltpu.async_copy(h_hbm.at[srcall.at[t0 + 1]], rows1, gs1)
        pltpu.make_async_copy(h_hbm.at[srcall.at[t0]], rows0, gs0).wait()
        pltpu.sync_copy(rows0, acc.at[dstall.at[t0]], add=True)
        pltpu.async_copy(h_hbm.at[srcall.at[t0 + 2]], rows0, gs0)
        pltpu.make_async_copy(h_hbm.at[srcall.at[t0 + 1]], rows1, gs1).wait()
        pltpu.sync_copy(rows1, acc.at[dstall.at[t0 + 1]], add=True)
        return carry

    lax.fori_loop(0, (K - 1) // 2, group, None)
    pltpu.make_async_copy(h_hbm.at[srcall.at[K - 1]], rows0, gs0).wait()
    pltpu.sync_copy(rows0, acc.at[dstall.at[K - 1]], add=True)
    plsc.subcore_barrier()

    pltpu.sync_copy(acc.at[pl.ds(s * RPS, RPS)],
                    out_hbm.at[pl.ds(c * N_PAD + s * RPS, RPS)])


@functools.cache
def _make_sc_agg(W):
    return pl.kernel(
        functools.partial(_sc_agg_body, W),
        out_type=jax.ShapeDtypeStruct((2 * N_PAD, W), jnp.float32),
        mesh=plsc.VectorSubcoreMesh(core_axis_name="c", subcore_axis_name="s",
                                    num_cores=NC, num_subcores=NS),
        scratch_types=[
            pltpu.VMEM((K, C), jnp.int32),
            pltpu.VMEM((K, C), jnp.int32),
            pltpu.VMEM((C, W), jnp.float32),
            pltpu.VMEM((C, W), jnp.float32),
            pltpu.VMEM_SHARED((N_PAD, W), jnp.float32),
            pltpu.SemaphoreType.DMA,
            pltpu.SemaphoreType.DMA,
        ],
        compiler_params=pltpu.CompilerParams(use_tc_tiling_on_sc=False),
    )


def _sc_agg(h, src, dst):
    return _make_sc_agg(h.shape[1])(h, src.reshape(TILES * K, C),
                                    dst.reshape(TILES * K, C))


def _h0_body(x_ref, eq_ref, tab_ref, o_ref):
    labels = eq_ref[0, 0, :]
    oh = (labels[:, None] ==
          lax.broadcasted_iota(jnp.int32, (1, NEQ), 1)).astype(jnp.float32)
    emb = jnp.dot(oh, tab_ref[...], preferred_element_type=jnp.float32,
                  precision=lax.Precision.HIGHEST)
    o_ref[...] = jnp.concatenate([x_ref[...], emb], axis=1)


def _h0(x, eq_labels, eq_table):
    eq3 = eq_labels.reshape(NB, 1, BN)
    return pl.pallas_call(
        _h0_body,
        grid=(NB,),
        in_specs=[
            pl.BlockSpec((BN, D), lambda b: (b, 0)),
            pl.BlockSpec((1, 1, BN), lambda b: (b, 0, 0)),
            pl.BlockSpec((NEQ, EMB), lambda b: (0, 0)),
        ],
        out_specs=pl.BlockSpec((BN, DIN), lambda b: (b, 0)),
        out_shape=jax.ShapeDtypeStruct((N, DIN), jnp.float32),
    )(x, eq3, eq_table)


def _layer_body(readout, eps_ref, h_ref, parts_ref, wa_ref, ba_ref, wb_ref,
                bb_ref, g_ref, be_ref, wf_ref, bf_ref, o_ref,
                z_scr, s_scr, ssq_scr):
    ph = pl.program_id(0)
    b = pl.program_id(1)

    @pl.when(ph == 0)
    def _mlp():
        agg = parts_ref[0] + parts_ref[1]
        z = (1.0 + eps_ref[0]) * h_ref[...] + agg
        a = jnp.maximum(
            jnp.dot(z, wa_ref[...], preferred_element_type=jnp.float32)
            + ba_ref[...], 0.0)
        z2 = jnp.maximum(
            jnp.dot(a, wb_ref[...], preferred_element_type=jnp.float32)
            + bb_ref[...], 0.0)
        z_scr[pl.ds(b * BN, BN), :] = z2

        @pl.when(b == 0)
        def _init_s():
            s_scr[...] = jnp.zeros_like(s_scr)

        s_scr[...] += jnp.sum(z2, axis=0, keepdims=True)

    @pl.when(ph == 1)
    def _var():
        @pl.when(b == 0)
        def _init_q():
            ssq_ref = ssq_scr
            ssq_ref[...] = jnp.zeros_like(ssq_ref)
        mu = s_scr[...] / N
        d = z_scr[pl.ds(b * BN, BN), :] - mu
        ssq_scr[...] += jnp.sum(d * d, axis=0, keepdims=True)

    @pl.when(ph == 2)
    def _norm():
        mu = s_scr[...] / N
        var = ssq_scr[...] / N
        zn = ((z_scr[pl.ds(b * BN, BN), :] - mu) / jnp.sqrt(var + 1e-5)
              * g_ref[...] + be_ref[...])
        hh = jnp.maximum(zn, 0.0)
        if readout:
            o_ref[...] = (jnp.dot(hh, wf_ref[...],
                                  preferred_element_type=jnp.float32)
                          + bf_ref[...])
        else:
            o_ref[...] = hh


def _layer(eps, h, parts, Wa, ba, Wb, bb, g, be, Wf, bf):
    Wd = h.shape[1]
    readout = Wf is not None
    dout = 1 if readout else H
    if not readout:
        Wf = jnp.zeros((1, 1), jnp.float32)
        bf = jnp.zeros((1,), jnp.float32)
    wf_shape = Wf.shape
    return pl.pallas_call(
        functools.partial(_layer_body, readout),
        grid=(3, NB),
        in_specs=[
            pl.BlockSpec(memory_space=pltpu.SMEM),
            pl.BlockSpec((BN, Wd), lambda ph, b: (jnp.where(ph == 0, b, 0), 0)),
            pl.BlockSpec((2, BN, Wd),
                         lambda ph, b: (0, jnp.where(ph == 0, b, 0), 0)),
            pl.BlockSpec((Wd, H), lambda ph, b: (0, 0)),
            pl.BlockSpec((1, H), lambda ph, b: (0, 0)),
            pl.BlockSpec((H, H), lambda ph, b: (0, 0)),
            pl.BlockSpec((1, H), lambda ph, b: (0, 0)),
            pl.BlockSpec((1, H), lambda ph, b: (0, 0)),
            pl.BlockSpec((1, H), lambda ph, b: (0, 0)),
            pl.BlockSpec(wf_shape, lambda ph, b: (0, 0)),
            pl.BlockSpec((1, 1), lambda ph, b: (0, 0)),
        ],
        out_specs=pl.BlockSpec((BN, dout),
                               lambda ph, b: (jnp.where(ph == 2, b, 0), 0)),
        out_shape=jax.ShapeDtypeStruct((N, dout), jnp.float32),
        scratch_shapes=[
            pltpu.VMEM((N, H), jnp.float32),
            pltpu.VMEM((1, H), jnp.float32),
            pltpu.VMEM((1, H), jnp.float32),
        ],
    )(jnp.reshape(eps, (1,)), h, parts.reshape(2, N_PAD, Wd), Wa,
      ba.reshape(1, H), Wb, bb.reshape(1, H), g.reshape(1, H),
      be.reshape(1, H), Wf, bf.reshape(1, 1))


def kernel(x, edge_index, eq_labels, batch, eq_table, eps0, W0a, b0a, W0b,
           b0b, g0, beta0, eps1, W1a, b1a, W1b, b1b, g1, beta1, eps2, W2a,
           b2a, W2b, b2b, g2, beta2, Wf, bf):
    src, dst = edge_index[0], edge_index[1]

    h = _h0(x, eq_labels, eq_table)

    layers = [
        (eps0, W0a, b0a, W0b, b0b, g0, beta0),
        (eps1, W1a, b1a, W1b, b1b, g1, beta1),
        (eps2, W2a, b2a, W2b, b2b, g2, beta2),
    ]

    for i, (eps, Wa, ba, Wb, bb, g, be) in enumerate(layers):
        parts = _sc_agg(h, src, dst)
        if i < 2:
            h = _layer(eps, h, parts, Wa, ba, Wb, bb, g, be, None, None)
        else:
            out = _layer(eps, h, parts, Wa, ba, Wb, bb, g, be, Wf, bf)

    return out.reshape(N)
